# baseline (speedup 1.0000x reference)
import functools

import jax
import jax.numpy as jnp
from jax import lax
from jax.experimental import pallas as pl
from jax.experimental.pallas import tpu as pltpu
from jax.experimental.pallas import tpu_sc as plsc

_CHUNK = 32


def _make_sc_kernel(batch, seq_len, dim, num_cores, num_subcores, chunk):
    num_workers = num_cores * num_subcores
    rows_per_worker = seq_len // num_workers
    n = rows_per_worker // chunk
    half = n // 2

    def body(w_hbm, out_hbm, vbuf, sbuf, sem_a, sem_b):
        cid = lax.axis_index("c")
        sid = lax.axis_index("s")
        wid = sid * num_cores + cid
        base = wid * rows_per_worker

        def rows(i):
            return pl.ds(base + i * chunk, chunk)

        def wave(src, i, sem):
            return [
                pltpu.make_async_copy(src, out_hbm.at[b, rows(i), :], sem)
                for b in range(batch)
            ]

        pltpu.sync_copy(w_hbm.at[rows(0), :], vbuf.at[0])
        pltpu.sync_copy(w_hbm.at[rows(1), :], sbuf.at[sid, 0])
        prev_a = wave(vbuf.at[0], 0, sem_a)
        prev_b = wave(sbuf.at[sid, 0], 1, sem_b)
        for cp in prev_a:
            cp.start()
        for cp in prev_b:
            cp.start()
        for p in range(1, half):
            k = p % 2
            pltpu.sync_copy(w_hbm.at[rows(2 * p), :], vbuf.at[k])
            pltpu.sync_copy(w_hbm.at[rows(2 * p + 1), :], sbuf.at[sid, k])
            for cp in prev_a:
                cp.wait()
            cur_a = wave(vbuf.at[k], 2 * p, sem_a)
            for cp in cur_a:
                cp.start()
            for cp in prev_b:
                cp.wait()
            cur_b = wave(sbuf.at[sid, k], 2 * p + 1, sem_b)
            for cp in cur_b:
                cp.start()
            prev_a, prev_b = cur_a, cur_b
        for cp in prev_a:
            cp.wait()
        for cp in prev_b:
            cp.wait()

    return body


def kernel(input_ids, emb_weight):
    batch, seq_len = input_ids.shape
    dim = emb_weight.shape[1]
    info = plsc.get_sparse_core_info()
    chunk = _CHUNK
    mesh = plsc.VectorSubcoreMesh(core_axis_name="c", subcore_axis_name="s")
    k = functools.partial(
        pl.kernel,
        mesh=mesh,
        out_type=jax.ShapeDtypeStruct((batch, seq_len, dim), emb_weight.dtype),
        scratch_types=[
            pltpu.VMEM((2, chunk, dim), jnp.float32),
            pltpu.MemorySpace.VMEM_SHARED(
                (info.num_subcores, 2, chunk, dim), jnp.float32
            ),
            pltpu.SemaphoreType.DMA,
            pltpu.SemaphoreType.DMA,
        ],
    )(
        _make_sc_kernel(
            batch, seq_len, dim, info.num_cores, info.num_subcores, chunk
        )
    )
    return k(emb_weight)


# speedup vs baseline: 1.0503x; 1.0503x over previous
import functools

import jax
import jax.numpy as jnp
from jax import lax
from jax.experimental import pallas as pl
from jax.experimental.pallas import tpu as pltpu
from jax.experimental.pallas import tpu_sc as plsc

_CHUNK = 32


def _make_sc_kernel(batch, seq_len, dim, num_cores, num_subcores, chunk):
    rows_per_worker = seq_len // (num_cores * num_subcores)
    n = rows_per_worker // chunk

    def body(w_hbm, out_hbm, buf, sem):
        wid = lax.axis_index("s") * num_cores + lax.axis_index("c")
        base = wid * rows_per_worker

        def rows(i):
            return pl.ds(base + i * chunk, chunk)

        pltpu.sync_copy(w_hbm.at[rows(0), :], buf.at[0])
        for i in range(n):
            copies = [
                pltpu.make_async_copy(
                    buf.at[i % 2], out_hbm.at[b, rows(i), :], sem
                )
                for b in range(batch)
            ]
            for c in copies:
                c.start()
            if i + 1 < n:
                pltpu.sync_copy(w_hbm.at[rows(i + 1), :], buf.at[(i + 1) % 2])
            for c in copies:
                c.wait()

    return body


def kernel(input_ids, emb_weight):
    batch, seq_len = input_ids.shape
    dim = emb_weight.shape[1]
    info = plsc.get_sparse_core_info()
    chunk = _CHUNK
    mesh = plsc.VectorSubcoreMesh(core_axis_name="c", subcore_axis_name="s")
    k = functools.partial(
        pl.kernel,
        mesh=mesh,
        out_type=jax.ShapeDtypeStruct((batch, seq_len, dim), emb_weight.dtype),
        scratch_types=[
            pltpu.VMEM((2, chunk, dim), jnp.float32),
            pltpu.SemaphoreType.DMA,
        ],
    )(
        _make_sc_kernel(
            batch, seq_len, dim, info.num_cores, info.num_subcores, chunk
        )
    )
    return k(emb_weight)
